# baseline (speedup 1.0000x reference)
import functools
import math

import jax
import jax.numpy as jnp
from jax import lax
from jax.experimental import pallas as pl
from jax.experimental.pallas import tpu as pltpu
from jax.experimental.pallas import tpu_sc as plsc

BLK = 128


def _routing_metadata(actor_idx, n, e, nblk):
    idx = actor_idx.astype(jnp.int32)
    oh = (idx[:, None] == jnp.arange(e, dtype=jnp.int32)[None, :]).astype(jnp.int32)
    rank = jnp.take_along_axis(jnp.cumsum(oh, axis=0) - oh, idx[:, None], axis=1)[:, 0]
    counts = jnp.sum(oh, axis=0)
    padded = ((counts + BLK - 1) // BLK) * BLK
    ends = jnp.cumsum(padded)
    starts = ends - padded
    dest = starts[idx] + rank
    src = jnp.zeros((nblk * BLK,), jnp.int32).at[dest].set(
        jnp.arange(n, dtype=jnp.int32))
    block_rows = jnp.arange(nblk, dtype=jnp.int32) * BLK
    block_expert = jnp.minimum(
        jnp.searchsorted(ends, block_rows, side="right"), e - 1).astype(jnp.int32)
    return src, dest, block_expert


def _sc_gather_in(state, eps, src, npad, d, ae):
    n = state.shape[0]
    info = plsc.get_sparse_core_info()
    nw = info.num_cores * info.num_subcores
    per_w = npad // nw
    ch = per_w // 4
    eh = per_w // 2

    mesh = plsc.VectorSubcoreMesh(core_axis_name="c", subcore_axis_name="s")

    @functools.partial(
        pl.kernel,
        mesh=mesh,
        out_type=(
            jax.ShapeDtypeStruct((npad, d), jnp.float32),
            jax.ShapeDtypeStruct((npad, ae), jnp.float32),
        ),
        scratch_types=[
            pltpu.VMEM((per_w,), jnp.int32),
            pltpu.VMEM((ch, d), jnp.float32),
            pltpu.VMEM((ch, d), jnp.float32),
            pltpu.VMEM((per_w, ae), jnp.float32),
            pltpu.SemaphoreType.DMA,
            pltpu.SemaphoreType.DMA,
            pltpu.SemaphoreType.DMA,
        ],
    )
    def gather_kernel(state_hbm, eps_hbm, src_hbm, xpad_hbm, epad_hbm,
                      idx_v, buf0, buf1, ebuf, sem0, sem1, esem):
        wid = lax.axis_index("s") * info.num_cores + lax.axis_index("c")
        base = wid * per_w
        pltpu.sync_copy(src_hbm.at[pl.ds(base, per_w)], idx_v)
        e0 = pltpu.async_copy(
            eps_hbm.at[idx_v.at[pl.ds(0, eh)]], ebuf.at[pl.ds(0, eh)], esem)
        g0 = pltpu.async_copy(
            state_hbm.at[idx_v.at[pl.ds(0 * ch, ch)]], buf0, sem0)
        g1 = pltpu.async_copy(
            state_hbm.at[idx_v.at[pl.ds(1 * ch, ch)]], buf1, sem1)
        g0.wait()
        w0 = pltpu.async_copy(buf0, xpad_hbm.at[pl.ds(base + 0 * ch, ch)], sem0)
        g1.wait()
        w1 = pltpu.async_copy(buf1, xpad_hbm.at[pl.ds(base + 1 * ch, ch)], sem1)
        e0.wait()
        e1 = pltpu.async_copy(
            eps_hbm.at[idx_v.at[pl.ds(eh, eh)]], ebuf.at[pl.ds(eh, eh)], esem)
        w0.wait()
        g2 = pltpu.async_copy(
            state_hbm.at[idx_v.at[pl.ds(2 * ch, ch)]], buf0, sem0)
        w1.wait()
        g3 = pltpu.async_copy(
            state_hbm.at[idx_v.at[pl.ds(3 * ch, ch)]], buf1, sem1)
        g2.wait()
        w2 = pltpu.async_copy(buf0, xpad_hbm.at[pl.ds(base + 2 * ch, ch)], sem0)
        g3.wait()
        w3 = pltpu.async_copy(buf1, xpad_hbm.at[pl.ds(base + 3 * ch, ch)], sem1)
        e1.wait()
        ew = pltpu.async_copy(ebuf, epad_hbm.at[pl.ds(base, per_w)], esem)
        w2.wait()
        w3.wait()
        ew.wait()

    return gather_kernel(state, eps, src)


def _tc_grouped_head(x_pad, eps_pad, W_mu, b_mu, W_ls, b_ls, block_expert,
                     npad, d, a):
    nblk = npad // BLK
    log2pi = math.log(2.0 * math.pi)

    def body(expert_ref, x_ref, wmu_ref, bmu_ref, wls_ref, bls_ref, eps_ref,
             y_ref):
        del expert_ref
        x = x_ref[...]
        mu = jnp.dot(x, wmu_ref[0], preferred_element_type=jnp.float32)
        mu = mu + bmu_ref[0]
        ls = jnp.dot(x, wls_ref[0], preferred_element_type=jnp.float32)
        ls = jnp.clip(ls + bls_ref[0], -5.0, 2.0)
        eps = eps_ref[...][:, :a]
        act = mu + jnp.exp(ls) * eps
        lp = (-jnp.sum(ls, axis=1, keepdims=True)
              - 0.5 * jnp.sum(eps * eps, axis=1, keepdims=True)
              - (0.5 * a * log2pi))
        y_ref[...] = jnp.concatenate(
            [act, jnp.broadcast_to(lp, (BLK, a))], axis=1)

    grid_spec = pltpu.PrefetchScalarGridSpec(
        num_scalar_prefetch=1,
        grid=(nblk,),
        in_specs=[
            pl.BlockSpec((BLK, d), lambda b, er: (b, 0)),
            pl.BlockSpec((1, d, a), lambda b, er: (er[b], 0, 0)),
            pl.BlockSpec((1, 1, a), lambda b, er: (er[b], 0, 0)),
            pl.BlockSpec((1, d, a), lambda b, er: (er[b], 0, 0)),
            pl.BlockSpec((1, 1, a), lambda b, er: (er[b], 0, 0)),
            pl.BlockSpec((BLK, 2 * a), lambda b, er: (b, 0)),
        ],
        out_specs=pl.BlockSpec((BLK, 2 * a), lambda b, er: (b, 0)),
    )
    return pl.pallas_call(
        body,
        grid_spec=grid_spec,
        out_shape=jax.ShapeDtypeStruct((npad, 2 * a), jnp.float32),
        compiler_params=pltpu.CompilerParams(
            dimension_semantics=("arbitrary",)),
    )(block_expert, x_pad, W_mu, b_mu.reshape(b_mu.shape[0], 1, a),
      W_ls, b_ls.reshape(b_ls.shape[0], 1, a), eps_pad)


def _sc_gather_out(y_pad, dest, n, w):
    info = plsc.get_sparse_core_info()
    nw = info.num_cores * info.num_subcores
    per_w = n // nw

    mesh = plsc.VectorSubcoreMesh(core_axis_name="c", subcore_axis_name="s")

    @functools.partial(
        pl.kernel,
        mesh=mesh,
        out_type=jax.ShapeDtypeStruct((n, w), jnp.float32),
        scratch_types=[
            pltpu.VMEM((per_w,), jnp.int32),
            pltpu.VMEM((per_w, w), jnp.float32),
            pltpu.SemaphoreType.DMA,
        ],
    )
    def gather_back(ypad_hbm, dest_hbm, out_hbm, idx_v, rows_v, sem):
        wid = lax.axis_index("s") * info.num_cores + lax.axis_index("c")
        base = wid * per_w
        pltpu.sync_copy(dest_hbm.at[pl.ds(base, per_w)], idx_v)
        pltpu.async_copy(ypad_hbm.at[idx_v], rows_v, sem).wait()
        pltpu.sync_copy(rows_v, out_hbm.at[pl.ds(base, per_w)])

    return gather_back(y_pad, dest)


def kernel(state, W_mu, b_mu, W_ls, b_ls, mix_weights):
    n, d = state.shape
    e, _, a = W_mu.shape
    npad = n + e * BLK
    nblk = npad // BLK

    actor_idx = jax.random.categorical(
        jax.random.fold_in(jax.random.key(1), 7), jnp.log(mix_weights),
        shape=(n,))
    eps = jax.random.normal(
        jax.random.fold_in(jax.random.key(1), 11), (n, a), dtype=state.dtype)
    eps_wide = jnp.pad(eps, ((0, 0), (0, a)))

    src, dest, block_expert = _routing_metadata(actor_idx, n, e, nblk)

    x_pad, eps_pad = _sc_gather_in(state, eps_wide, src, npad, d, 2 * a)
    y_pad = _tc_grouped_head(x_pad, eps_pad, W_mu, b_mu, W_ls, b_ls,
                             block_expert, npad, d, a)
    y = _sc_gather_out(y_pad, dest, n, 2 * a)
    return y[:, :a], y[:, a]
